# baseline (speedup 1.0000x reference)
import functools

import jax
import jax.numpy as jnp
from jax import lax
from jax.experimental import pallas as pl
from jax.experimental.pallas import tpu as pltpu
from jax.experimental.pallas import tpu_sc as plsc

_B = 16384
_D = 768
_V = 12

_info = plsc.get_sparse_core_info()
_NC = _info.num_cores
_NS = _info.num_subcores
_NW = _NC * _NS
_BPW = _B // _NW
_L = 16
_Q = 16

_mesh = plsc.VectorSubcoreMesh(core_axis_name="c", subcore_axis_name="s")


@functools.partial(
    pl.kernel,
    mesh=_mesh,
    out_type=jax.ShapeDtypeStruct((_B * _D,), jnp.float32),
    scratch_types=[
        pltpu.VMEM((_BPW + _L,), jnp.int32),
        pltpu.VMEM((_V * _D,), jnp.float32),
        pltpu.SemaphoreType.DMA,
        pltpu.SemaphoreType.DMA,
    ],
    compiler_params=pltpu.CompilerParams(
        use_tc_tiling_on_sc=False, needs_layout_passes=False
    ),
)
def _emb_lookup(labels_hbm, table_hbm, out_hbm, idx_v, table_v, ssem, stsem):
    wid = lax.axis_index("s") * _NC + lax.axis_index("c")
    base = wid * _BPW
    tcp = pltpu.make_async_copy(table_hbm, table_v, stsem)
    lcp = pltpu.make_async_copy(
        labels_hbm.at[pl.ds(base, _BPW)], idx_v.at[pl.ds(0, _BPW)], stsem
    )
    tcp.start()
    lcp.start()
    tcp.wait()
    lcp.wait()

    def row_copy(r):
        lab = idx_v[pl.ds(r, _L)][0]
        return pltpu.make_async_copy(
            table_v.at[pl.ds(lab * _D, _D)],
            out_hbm.at[pl.ds((base + r) * _D, _D)],
            ssem,
        )

    for r in range(_Q):
        row_copy(r).start()

    def body(r, _):
        row_copy(r + _Q).start()
        row_copy(r).wait()
        return 0

    lax.fori_loop(0, _BPW - _Q, body, 0)
    for r in range(_BPW - _Q, _BPW):
        row_copy(r).wait()


def kernel(labels, table):
    out = _emb_lookup(labels.astype(jnp.int32), table.reshape(-1))
    return out.reshape(_B, _D)


# speedup vs baseline: 1.0028x; 1.0028x over previous
import functools

import jax
import jax.numpy as jnp
from jax import lax
from jax.experimental import pallas as pl
from jax.experimental.pallas import tpu as pltpu
from jax.experimental.pallas import tpu_sc as plsc

_B = 16384
_D = 768
_V = 12

_info = plsc.get_sparse_core_info()
_NC = _info.num_cores
_NS = _info.num_subcores
_NW = _NC * _NS
_BPW = _B // _NW
_L = 16
_Q = 8

_mesh = plsc.VectorSubcoreMesh(core_axis_name="c", subcore_axis_name="s")


@functools.partial(
    pl.kernel,
    mesh=_mesh,
    out_type=jax.ShapeDtypeStruct((_B * _D,), jnp.float32),
    scratch_types=[
        pltpu.VMEM((_BPW + _L,), jnp.int32),
        pltpu.VMEM((_V * _D,), jnp.float32),
        pltpu.SemaphoreType.DMA,
        pltpu.SemaphoreType.DMA,
    ],
    compiler_params=pltpu.CompilerParams(
        use_tc_tiling_on_sc=False, needs_layout_passes=False
    ),
)
def _emb_lookup(labels_hbm, table_hbm, out_hbm, idx_v, table_v, ssem, stsem):
    wid = lax.axis_index("s") * _NC + lax.axis_index("c")
    base = wid * _BPW
    tcp = pltpu.make_async_copy(table_hbm, table_v, stsem)
    lcp = pltpu.make_async_copy(
        labels_hbm.at[pl.ds(base, _BPW)], idx_v.at[pl.ds(0, _BPW)], stsem
    )
    tcp.start()
    lcp.start()
    tcp.wait()
    lcp.wait()

    def row_copy(r):
        lab = idx_v[pl.ds(r, _L)][0]
        return pltpu.make_async_copy(
            table_v.at[pl.ds(lab * _D, _D)],
            out_hbm.at[pl.ds((base + r) * _D, _D)],
            ssem,
        )

    for r in range(_Q):
        row_copy(r).start()

    def body(r, _):
        row_copy(r + _Q).start()
        row_copy(r).wait()
        return 0

    lax.fori_loop(0, _BPW - _Q, body, 0)
    for r in range(_BPW - _Q, _BPW):
        row_copy(r).wait()


def kernel(labels, table):
    out = _emb_lookup(labels.astype(jnp.int32), table.reshape(-1))
    return out.reshape(_B, _D)


# speedup vs baseline: 1.0279x; 1.0250x over previous
import functools

import jax
import jax.numpy as jnp
from jax import lax
from jax.experimental import pallas as pl
from jax.experimental.pallas import tpu as pltpu
from jax.experimental.pallas import tpu_sc as plsc

_B = 16384
_D = 768
_V = 12

_info = plsc.get_sparse_core_info()
_NC = _info.num_cores
_NS = _info.num_subcores
_NW = _NC * _NS
_BPW = _B // _NW
_L = 16
_Q = 8

_mesh = plsc.VectorSubcoreMesh(core_axis_name="c", subcore_axis_name="s")


@functools.partial(
    pl.kernel,
    mesh=_mesh,
    out_type=jax.ShapeDtypeStruct((_B * _D,), jnp.float32),
    scratch_types=[
        pltpu.VMEM((_BPW + _L,), jnp.int32),
        pltpu.VMEM((_V * _D,), jnp.float32),
        pltpu.SemaphoreType.DMA,
    ],
    compiler_params=pltpu.CompilerParams(
        use_tc_tiling_on_sc=False, needs_layout_passes=False
    ),
)
def _emb_lookup(labels_hbm, table_hbm, out_hbm, idx_v, table_v, ssem):
    wid = lax.axis_index("s") * _NC + lax.axis_index("c")
    base = wid * _BPW
    pltpu.sync_copy(table_hbm, table_v)
    pltpu.sync_copy(labels_hbm.at[pl.ds(base, _BPW)], idx_v.at[pl.ds(0, _BPW)])

    def row_copy(r):
        lab = idx_v[pl.ds(r, _L)][0]
        return pltpu.make_async_copy(
            table_v.at[pl.ds(lab * _D, _D)],
            out_hbm.at[pl.ds((base + r) * _D, _D)],
            ssem,
        )

    for r in range(_Q):
        row_copy(r).start()

    def body(r, _):
        row_copy(r + _Q).start()
        row_copy(r).wait()
        return 0

    lax.fori_loop(0, _BPW - _Q, body, 0)
    for r in range(_BPW - _Q, _BPW):
        row_copy(r).wait()


def kernel(labels, table):
    out = _emb_lookup(labels.astype(jnp.int32), table.reshape(-1))
    return out.reshape(_B, _D)
